# baseline (speedup 1.0000x reference)
import functools

import numpy as np
import jax
import jax.numpy as jnp
from jax import lax
from jax.experimental import pallas as pl
from jax.experimental.pallas import tpu as pltpu
from jax.experimental.pallas import tpu_sc as plsc

_N = 10000
_E = 160000
_H = 128
_NC = 2
_NS = 16
_NW = _NC * _NS
_CH = 40
_NCHUNKS = _E // _CH

_EBLK = 2000
_NBLK = 2000
_F32 = jnp.float32


def _lnk(x, g, b, eps=1e-5):
    m = jnp.mean(x, axis=-1, keepdims=True)
    v = jnp.mean((x - m) * (x - m), axis=-1, keepdims=True)
    return (x - m) / jnp.sqrt(v + eps) * g + b


def _dot(a, b):
    return jnp.dot(a, b, preferred_element_type=_F32)


def _node_embed_body(x_ref, dt_ref, ph_ref, wt_ref, b_ref, o_ref):
    x = x_ref[...]
    dt = dt_ref[...]
    ph = ph_ref[...]
    two_pi = np.float32(2.0 * np.pi)
    py = jnp.sin(x[:, 0:1] * two_pi / dt + ph)
    px = jnp.sin(x[:, 1:2] * two_pi / dt + ph)
    emb = jnp.concatenate([py, px], axis=1)
    o_ref[...] = _dot(emb, wt_ref[...]) + b_ref[...]


def _edge_embed_body(e_ref, dt_ref, ph_ref, wt_ref, b_ref, w4_ref, b4_ref,
                     ee_ref, ce_ref):
    emb = jnp.sin(e_ref[...] / dt_ref[...] + ph_ref[...])
    ee = _dot(emb, wt_ref[...]) + b_ref[...]
    ee_ref[...] = ee
    ce_ref[...] = _dot(ee, w4_ref[...]) + b4_ref[...]


def _nodeproj_body(h_ref, wu_ref, bu_ref, wa_ref, ba_ref, wb_ref, bb_ref,
                   wv_ref, bv_ref, uh_ref, av_ref, bh_ref):
    h = h_ref[...]
    uh_ref[...] = _dot(h, wu_ref[...]) + bu_ref[...]
    av_ref[:, 0:_H] = _dot(h, wa_ref[...]) + ba_ref[...]
    av_ref[:, _H:2 * _H] = _dot(h, wv_ref[...]) + bv_ref[...]
    bh_ref[...] = _dot(h, wb_ref[...]) + bb_ref[...]


def _passB_body(en_ref, ein_ref, ge_ref, be_ref, gp_ref, bp_ref, pw_ref,
                pb2_ref, w4_ref, b4_ref, ee_ref, ce_ref):
    x = jnp.maximum(_lnk(en_ref[...], ge_ref[...], be_ref[...]), 0.0)
    t = _lnk(x, gp_ref[...], bp_ref[...])
    t = t * jax.nn.sigmoid(t)
    ee = ein_ref[...] + _dot(t, pw_ref[...]) + pb2_ref[...]
    ee_ref[...] = ee
    ce_ref[...] = _dot(ee, w4_ref[...]) + b4_ref[...]


def _passB_last_body(en_ref, ein_ref, ge_ref, be_ref, gp_ref, bp_ref, pw_ref,
                     pb2_ref, ee_ref, mom_ref):
    x = jnp.maximum(_lnk(en_ref[...], ge_ref[...], be_ref[...]), 0.0)
    t = _lnk(x, gp_ref[...], bp_ref[...])
    t = t * jax.nn.sigmoid(t)
    ee = ein_ref[...] + _dot(t, pw_ref[...]) + pb2_ref[...]
    ee_ref[...] = ee

    @pl.when(pl.program_id(0) == 0)
    def _():
        mom_ref[...] = jnp.zeros_like(mom_ref)

    mom_ref[0:1, :] = mom_ref[0:1, :] + jnp.sum(ee, axis=0, keepdims=True)
    mom_ref[1:2, :] = mom_ref[1:2, :] + jnp.sum(ee * ee, axis=0, keepdims=True)


def _passC_body(h_ref, uh_ref, a0_ref, a1_ref, gh_ref, bh_ref, wu_ref, bu_ref,
                wa_ref, ba_ref, wb_ref, bb_ref, wv_ref, bv_ref,
                ho_ref, uho_ref, avo_ref, bho_ref):
    s = uh_ref[...] + a0_ref[...] + a1_ref[...]
    hn = jnp.maximum(_lnk(s, gh_ref[...], bh_ref[...]), 0.0)
    h = h_ref[...] + hn
    ho_ref[...] = h
    uho_ref[...] = _dot(h, wu_ref[...]) + bu_ref[...]
    avo_ref[:, 0:_H] = _dot(h, wa_ref[...]) + ba_ref[...]
    avo_ref[:, _H:2 * _H] = _dot(h, wv_ref[...]) + bv_ref[...]
    bho_ref[...] = _dot(h, wb_ref[...]) + bb_ref[...]


def _passC_last_body(h_ref, uh_ref, a0_ref, a1_ref, gh_ref, bh_ref, ho_ref):
    s = uh_ref[...] + a0_ref[...] + a1_ref[...]
    hn = jnp.maximum(_lnk(s, gh_ref[...], bh_ref[...]), 0.0)
    ho_ref[...] = h_ref[...] + hn


def _final_body(ee_ref, a_ref, s_ref, cw_ref, cb_ref, o_ref):
    y = jnp.maximum(ee_ref[...] * a_ref[...] + s_ref[...], 0.0)
    o_ref[...] = _dot(y, cw_ref[...]) + cb_ref[...]


def _full(shape):
    return pl.BlockSpec(shape, lambda i: (0,) * len(shape))


def _rows(blk, width):
    return pl.BlockSpec((blk, width), lambda i: (i, 0))


def _node_embed(x, dt64, ph64, wt, b):
    grid = (_N // _NBLK,)
    return pl.pallas_call(
        _node_embed_body,
        grid=grid,
        in_specs=[_rows(_NBLK, 2), _full((1, 64)), _full((1, 64)),
                  _full((_H, _H)), _full((1, _H))],
        out_specs=_rows(_NBLK, _H),
        out_shape=jax.ShapeDtypeStruct((_N, _H), _F32),
    )(x, dt64, ph64, wt, b)


def _edge_embed(e2, dt128, ph128, wt, b, w4, b4):
    grid = (_E // _EBLK,)
    return pl.pallas_call(
        _edge_embed_body,
        grid=grid,
        in_specs=[_rows(_EBLK, 1), _full((1, _H)), _full((1, _H)),
                  _full((_H, _H)), _full((1, _H)), _full((_H, _H)),
                  _full((1, _H))],
        out_specs=[_rows(_EBLK, _H), _rows(_EBLK, _H)],
        out_shape=[jax.ShapeDtypeStruct((_E, _H), _F32),
                   jax.ShapeDtypeStruct((_E, _H), _F32)],
    )(e2, dt128, ph128, wt, b, w4, b4)


def _nodeproj(h, wu, bu, wa, ba, wb, bb, wv, bv):
    grid = (_N // _NBLK,)
    wspecs = [_full((_H, _H)), _full((1, _H))] * 4
    return pl.pallas_call(
        _nodeproj_body,
        grid=grid,
        in_specs=[_rows(_NBLK, _H)] + wspecs,
        out_specs=[_rows(_NBLK, _H), _rows(_NBLK, 2 * _H), _rows(_NBLK, _H)],
        out_shape=[jax.ShapeDtypeStruct((_N, _H), _F32),
                   jax.ShapeDtypeStruct((_N, 2 * _H), _F32),
                   jax.ShapeDtypeStruct((_N, _H), _F32)],
    )(h, wu, bu, wa, ba, wb, bb, wv, bv)


def _passB(e_new, ee_in, ge, be, gp, bp, pw, pb2, w4, b4):
    grid = (_E // _EBLK,)
    return pl.pallas_call(
        _passB_body,
        grid=grid,
        in_specs=[_rows(_EBLK, _H), _rows(_EBLK, _H),
                  _full((1, _H)), _full((1, _H)), _full((1, _H)),
                  _full((1, _H)), _full((_H, _H)), _full((1, _H)),
                  _full((_H, _H)), _full((1, _H))],
        out_specs=[_rows(_EBLK, _H), _rows(_EBLK, _H)],
        out_shape=[jax.ShapeDtypeStruct((_E, _H), _F32),
                   jax.ShapeDtypeStruct((_E, _H), _F32)],
    )(e_new, ee_in, ge, be, gp, bp, pw, pb2, w4, b4)


def _passB_last(e_new, ee_in, ge, be, gp, bp, pw, pb2):
    grid = (_E // _EBLK,)
    return pl.pallas_call(
        _passB_last_body,
        grid=grid,
        in_specs=[_rows(_EBLK, _H), _rows(_EBLK, _H),
                  _full((1, _H)), _full((1, _H)), _full((1, _H)),
                  _full((1, _H)), _full((_H, _H)), _full((1, _H))],
        out_specs=[_rows(_EBLK, _H), _full((8, _H))],
        out_shape=[jax.ShapeDtypeStruct((_E, _H), _F32),
                   jax.ShapeDtypeStruct((8, _H), _F32)],
    )(e_new, ee_in, ge, be, gp, bp, pw, pb2)


def _passC(h, uh, agg, gh, bh, wu, bu, wa, ba, wb, bb, wv, bv):
    grid = (_N // _NBLK,)
    nblocks = _N // _NBLK
    a1_spec = pl.BlockSpec((_NBLK, _H), lambda i: (i + nblocks, 0))
    wspecs = [_full((_H, _H)), _full((1, _H))] * 4
    return pl.pallas_call(
        _passC_body,
        grid=grid,
        in_specs=[_rows(_NBLK, _H), _rows(_NBLK, _H), _rows(_NBLK, _H),
                  a1_spec, _full((1, _H)), _full((1, _H))] + wspecs,
        out_specs=[_rows(_NBLK, _H), _rows(_NBLK, _H), _rows(_NBLK, 2 * _H),
                   _rows(_NBLK, _H)],
        out_shape=[jax.ShapeDtypeStruct((_N, _H), _F32),
                   jax.ShapeDtypeStruct((_N, _H), _F32),
                   jax.ShapeDtypeStruct((_N, 2 * _H), _F32),
                   jax.ShapeDtypeStruct((_N, _H), _F32)],
    )(h, uh, agg, agg, gh, bh, wu, bu, wa, ba, wb, bb, wv, bv)


def _passC_last(h, uh, agg, gh, bh):
    grid = (_N // _NBLK,)
    nblocks = _N // _NBLK
    a1_spec = pl.BlockSpec((_NBLK, _H), lambda i: (i + nblocks, 0))
    return pl.pallas_call(
        _passC_last_body,
        grid=grid,
        in_specs=[_rows(_NBLK, _H), _rows(_NBLK, _H), _rows(_NBLK, _H),
                  a1_spec, _full((1, _H)), _full((1, _H))],
        out_specs=_rows(_NBLK, _H),
        out_shape=jax.ShapeDtypeStruct((_N, _H), _F32),
    )(h, uh, agg, agg, gh, bh)


def _final(ee, a, s, cw, cb):
    grid = (_E // _EBLK,)
    return pl.pallas_call(
        _final_body,
        grid=grid,
        in_specs=[_rows(_EBLK, _H), _full((1, _H)), _full((1, _H)),
                  _full((_H, 8)), _full((1, 8))],
        out_specs=_rows(_EBLK, 8),
        out_shape=jax.ShapeDtypeStruct((_E, 8), _F32),
    )(ee, a, s, cw, cb)


_ZROWS = 40
_NZCHUNKS = _N // _ZROWS


_NLOC = _NCHUNKS // _NW


@functools.lru_cache(maxsize=1)
def _build_sc_edge():
    mesh = plsc.VectorSubcoreMesh(core_axis_name="c", subcore_axis_name="s",
                                  num_cores=_NC, num_subcores=_NS)

    scratch = (
        [pltpu.VMEM((_CH,), jnp.int32) for _ in range(4)]
        + [pltpu.VMEM((_CH,), jnp.int32) for _ in range(4)]
        + [pltpu.VMEM((_CH, 2 * _H), _F32) for _ in range(2)]
        + [pltpu.VMEM((_CH, _H), _F32) for _ in range(2)]
        + [pltpu.VMEM((_CH, _H), _F32),
           pltpu.VMEM((_CH, _H), _F32),
           pltpu.VMEM((_CH, _H), _F32),
           pltpu.VMEM_SHARED((_N, _H), _F32)]
        + [pltpu.SemaphoreType.DMA for _ in range(11)]
    )

    @functools.partial(
        pl.kernel,
        out_type=(
            jax.ShapeDtypeStruct((_E, _H), _F32),
            jax.ShapeDtypeStruct((_NC * _N, _H), _F32),
        ),
        mesh=mesh,
        scratch_types=scratch,
    )
    def sc_edge(av_hbm, bh_hbm, ce_hbm, dst_hbm, src_hbm,
                enew_hbm, agg_hbm,
                d0, d1, d2, d3, s0, s1, s2, s3,
                av0, av1, bh0, bh1, cev, env, gvv, agg_sh,
                id0, id1, is0, is1, ga0, ga1, gb0, gb1, gc, we, ws):
        c = lax.axis_index("c")
        s = lax.axis_index("s")
        wid = s * _NC + c

        dstv = [d0, d1, d2, d3]
        srcv = [s0, s1, s2, s3]
        avv = [av0, av1]
        bhv = [bh0, bh1]
        isem_d = [id0, id1]
        isem_s = [is0, is1]
        gsem_a = [ga0, ga1]
        gsem_b = [gb0, gb1]

        zvec = jnp.zeros((16,), _F32)

        def zfill(r, carry):
            for i in range(_H // 16):
                gvv[r, pl.ds(i * 16, 16)] = zvec
            return carry

        lax.fori_loop(0, _ZROWS, zfill, 0)

        nz = (_NZCHUNKS - s + _NS - 1) // _NS

        def zcopy(k, carry):
            g = s + k * _NS
            pltpu.sync_copy(gvv, agg_sh.at[pl.ds(g * _ZROWS, _ZROWS)])
            return carry

        lax.fori_loop(0, nz, zcopy, 0)
        plsc.subcore_barrier()

        def base(i):
            return (wid + i * _NW) * _CH

        def issue_idx(i, slot, sync):
            b = base(i)
            if sync:
                pltpu.sync_copy(dst_hbm.at[pl.ds(b, _CH)], dstv[slot])
                pltpu.sync_copy(src_hbm.at[pl.ds(b, _CH)], srcv[slot])
            else:
                pltpu.async_copy(dst_hbm.at[pl.ds(b, _CH)], dstv[slot],
                                 isem_d[slot % 2])
                pltpu.async_copy(src_hbm.at[pl.ds(b, _CH)], srcv[slot],
                                 isem_s[slot % 2])

        def wait_idx(i, slot):
            b = base(i)
            pltpu.make_async_copy(dst_hbm.at[pl.ds(b, _CH)], dstv[slot],
                                  isem_d[slot % 2]).wait()
            pltpu.make_async_copy(src_hbm.at[pl.ds(b, _CH)], srcv[slot],
                                  isem_s[slot % 2]).wait()

        def issue_gathers(i, p, slot):
            pltpu.async_copy(av_hbm.at[dstv[slot]], avv[p], gsem_a[p])
            pltpu.async_copy(bh_hbm.at[srcv[slot]], bhv[p], gsem_b[p])

        def wait_gathers(i, p, slot):
            pltpu.make_async_copy(av_hbm.at[dstv[slot]], avv[p],
                                  gsem_a[p]).wait()
            pltpu.make_async_copy(bh_hbm.at[srcv[slot]], bhv[p],
                                  gsem_b[p]).wait()

        def issue_ce(i):
            pltpu.async_copy(ce_hbm.at[pl.ds(base(i), _CH)], cev, gc)

        def wait_ce(i):
            pltpu.make_async_copy(ce_hbm.at[pl.ds(base(i), _CH)], cev,
                                  gc).wait()

        def issue_writes(i, slot):
            pltpu.async_copy(env, enew_hbm.at[pl.ds(base(i), _CH)], we)
            pltpu.async_copy(gvv, agg_sh.at[srcv[slot]], ws, add=True)

        def wait_writes(i, slot):
            pltpu.make_async_copy(env, enew_hbm.at[pl.ds(base(i), _CH)],
                                  we).wait()
            pltpu.make_async_copy(gvv, agg_sh.at[srcv[slot]], ws).wait()

        def compute(p):
            def row(j, carry2):
                for i2 in range(_H // 16):
                    a = avv[p][j, pl.ds(i2 * 16, 16)]
                    v = avv[p][j, pl.ds(_H + i2 * 16, 16)]
                    bb = bhv[p][j, pl.ds(i2 * 16, 16)]
                    cc = cev[j, pl.ds(i2 * 16, 16)]
                    en = a + bb + cc
                    env[j, pl.ds(i2 * 16, 16)] = en
                    gvv[j, pl.ds(i2 * 16, 16)] = v / (1.0 + jnp.exp(-en))
                return carry2

            lax.fori_loop(0, _CH, row, 0, unroll=2)

        issue_idx(0, 0, sync=True)
        issue_idx(1, 1, sync=False)
        issue_gathers(0, 0, 0)
        issue_ce(0)

        def quad(k, carry):
            for j in range(4):
                i = 4 * k + j
                p = j % 2

                @pl.when(jnp.logical_and(i >= 1, i <= _NLOC))
                def _():
                    wait_writes(i - 1, (j - 1) % 4)

                @pl.when(i + 2 <= _NLOC - 1)
                def _():
                    issue_idx(i + 2, (j + 2) % 4, sync=False)

                @pl.when(i + 1 <= _NLOC - 1)
                def _():
                    wait_idx(i + 1, (j + 1) % 4)
                    issue_gathers(i + 1, 1 - p, (j + 1) % 4)

                @pl.when(i <= _NLOC - 1)
                def _():
                    wait_gathers(i, p, j)
                    wait_ce(i)
                    compute(p)
                    issue_writes(i, j)

                @pl.when(i + 1 <= _NLOC - 1)
                def _():
                    issue_ce(i + 1)
            return carry

        lax.fori_loop(0, (_NLOC + 4) // 4, quad, 0)
        plsc.subcore_barrier()

        def ocopy(k, carry):
            g = s + k * _NS
            pltpu.sync_copy(agg_sh.at[pl.ds(g * _ZROWS, _ZROWS)],
                            agg_hbm.at[pl.ds(c * _N + g * _ZROWS, _ZROWS)])
            return carry

        lax.fori_loop(0, nz, ocopy, 0)

    return sc_edge


def _sc_edge(av, bh, ce, dst, src):
    return _build_sc_edge()(av, bh, ce, dst, src)


def kernel(task, x, e, edge_index, node_W, node_b, edge_W, edge_b, lin_W,
           lin_b, ln_h_g, ln_h_b, ln_e_g, ln_e_b, plo_g, plo_b, plo_W,
           plo_b2, gn_g, gn_b, conv_W, conv_b):
    src = edge_index[0]
    dst = edge_index[1]

    i64 = np.arange(64, dtype=np.float64)
    dt64 = (10000.0 ** (2.0 * np.floor(i64 / 2.0) / 64.0))
    dt64 = jnp.asarray(dt64.reshape(1, 64), dtype=_F32)
    ph64 = jnp.asarray(((i64 % 2.0) * (np.pi / 2.0)).reshape(1, 64), _F32)
    i128 = np.arange(128, dtype=np.float64)
    dt128 = (10000.0 ** (2.0 * np.floor(i128 / 2.0) / 128.0))
    dt128 = jnp.asarray(dt128.reshape(1, 128), dtype=_F32)
    ph128 = jnp.asarray(((i128 % 2.0) * (np.pi / 2.0)).reshape(1, 128), _F32)

    r1 = lambda a: a.reshape(1, _H)
    w4t = [lin_W[l, 4].T for l in range(4)]
    b4 = [r1(lin_b[l, 4]) for l in range(4)]
    wut = [lin_W[l, 0].T for l in range(4)]
    but = [r1(lin_b[l, 0]) for l in range(4)]
    wvt = [lin_W[l, 1].T for l in range(4)]
    bvt = [r1(lin_b[l, 1]) for l in range(4)]
    wat = [lin_W[l, 2].T for l in range(4)]
    bat = [r1(lin_b[l, 2]) for l in range(4)]
    wbt = [lin_W[l, 3].T for l in range(4)]
    bbt = [r1(lin_b[l, 3]) for l in range(4)]
    pwt = [plo_W[l].T for l in range(4)]

    h = _node_embed(x, dt64, ph64, node_W.T, r1(node_b))
    ee, ce = _edge_embed(e.reshape(_E, 1), dt128, ph128, edge_W.T,
                         r1(edge_b), w4t[0], b4[0])
    uh, av, bh = _nodeproj(h, wut[0], but[0], wat[0], bat[0],
                           wbt[0], bbt[0], wvt[0], bvt[0])

    moments = None
    for l in range(4):
        e_new, agg = _sc_edge(av, bh, ce, dst, src)
        if l < 3:
            ee, ce = _passB(e_new, ee, r1(ln_e_g[l]), r1(ln_e_b[l]),
                            r1(plo_g[l]), r1(plo_b[l]), pwt[l],
                            r1(plo_b2[l]), w4t[l + 1], b4[l + 1])
            h, uh, av, bh = _passC(h, uh, agg, r1(ln_h_g[l]), r1(ln_h_b[l]),
                                   wut[l + 1], but[l + 1], wat[l + 1],
                                   bat[l + 1], wbt[l + 1], bbt[l + 1],
                                   wvt[l + 1], bvt[l + 1])
        else:
            ee, moments = _passB_last(e_new, ee, r1(ln_e_g[l]), r1(ln_e_b[l]),
                                      r1(plo_g[l]), r1(plo_b[l]), pwt[l],
                                      r1(plo_b2[l]))
            h = _passC_last(h, uh, agg, r1(ln_h_g[l]), r1(ln_h_b[l]))

    csum = moments[0]
    csumsq = moments[1]
    cnt = np.float32(4 * _E)
    gmean = csum.reshape(32, 4).sum(axis=1) / cnt
    gsq = csumsq.reshape(32, 4).sum(axis=1) / cnt
    gvar = gsq - gmean * gmean
    ginv = 1.0 / jnp.sqrt(gvar + 1e-5)
    mean128 = jnp.repeat(gmean, 4)
    inv128 = jnp.repeat(ginv, 4)
    a = (inv128 * gn_g).reshape(1, _H)
    sh = (gn_b - mean128 * inv128 * gn_g).reshape(1, _H)
    cw = jnp.zeros((_H, 8), _F32).at[:, 0:2].set(conv_W.T)
    cb = jnp.zeros((1, 8), _F32).at[0, 0:2].set(conv_b)
    e_out8 = _final(ee, a, sh, cw, cb)
    e_out = e_out8[:, 0:2]
    return h, e_out


# speedup vs baseline: 2.2966x; 2.2966x over previous
import functools

import numpy as np
import jax
import jax.numpy as jnp
from jax import lax
from jax.experimental import pallas as pl
from jax.experimental.pallas import tpu as pltpu
from jax.experimental.pallas import tpu_sc as plsc

_N = 10000
_E = 160000
_H = 128
_NC = 2
_NS = 16
_NW = _NC * _NS
_CH = 40
_NCHUNKS = _E // _CH

_EBLK = 2000
_NBLK = 2000
_F32 = jnp.float32


def _lnk(x, g, b, eps=1e-5):
    m = jnp.mean(x, axis=-1, keepdims=True)
    v = jnp.mean((x - m) * (x - m), axis=-1, keepdims=True)
    return (x - m) / jnp.sqrt(v + eps) * g + b


def _dot(a, b):
    return jnp.dot(a, b, preferred_element_type=_F32)


def _node_embed_body(x_ref, dt_ref, ph_ref, wt_ref, b_ref, o_ref):
    x = x_ref[...]
    dt = dt_ref[...]
    ph = ph_ref[...]
    two_pi = np.float32(2.0 * np.pi)
    py = jnp.sin(x[:, 0:1] * two_pi / dt + ph)
    px = jnp.sin(x[:, 1:2] * two_pi / dt + ph)
    emb = jnp.concatenate([py, px], axis=1)
    o_ref[...] = _dot(emb, wt_ref[...]) + b_ref[...]


def _edge_embed_body(e_ref, dt_ref, ph_ref, wt_ref, b_ref, w4_ref, b4_ref,
                     ee_ref, ce_ref):
    emb = jnp.sin(e_ref[...] / dt_ref[...] + ph_ref[...])
    ee = _dot(emb, wt_ref[...]) + b_ref[...]
    ee_ref[...] = ee
    ce_ref[...] = _dot(ee, w4_ref[...]) + b4_ref[...]


def _nodeproj_body(h_ref, wu_ref, bu_ref, wa_ref, ba_ref, wb_ref, bb_ref,
                   wv_ref, bv_ref, uh_ref, av_ref, bh_ref):
    h = h_ref[...]
    uh_ref[...] = _dot(h, wu_ref[...]) + bu_ref[...]
    av_ref[:, 0:_H] = _dot(h, wa_ref[...]) + ba_ref[...]
    av_ref[:, _H:2 * _H] = _dot(h, wv_ref[...]) + bv_ref[...]
    bh_ref[...] = _dot(h, wb_ref[...]) + bb_ref[...]


def _passB_body(en_ref, ein_ref, ge_ref, be_ref, gp_ref, bp_ref, pw_ref,
                pb2_ref, w4_ref, b4_ref, ee_ref, ce_ref):
    x = jnp.maximum(_lnk(en_ref[...], ge_ref[...], be_ref[...]), 0.0)
    t = _lnk(x, gp_ref[...], bp_ref[...])
    t = t * jax.nn.sigmoid(t)
    ee = ein_ref[...] + _dot(t, pw_ref[...]) + pb2_ref[...]
    ee_ref[...] = ee
    ce_ref[...] = _dot(ee, w4_ref[...]) + b4_ref[...]


def _passB_last_body(en_ref, ein_ref, ge_ref, be_ref, gp_ref, bp_ref, pw_ref,
                     pb2_ref, ee_ref, mom_ref):
    x = jnp.maximum(_lnk(en_ref[...], ge_ref[...], be_ref[...]), 0.0)
    t = _lnk(x, gp_ref[...], bp_ref[...])
    t = t * jax.nn.sigmoid(t)
    ee = ein_ref[...] + _dot(t, pw_ref[...]) + pb2_ref[...]
    ee_ref[...] = ee

    @pl.when(pl.program_id(0) == 0)
    def _():
        mom_ref[...] = jnp.zeros_like(mom_ref)

    mom_ref[0:1, :] = mom_ref[0:1, :] + jnp.sum(ee, axis=0, keepdims=True)
    mom_ref[1:2, :] = mom_ref[1:2, :] + jnp.sum(ee * ee, axis=0, keepdims=True)


def _passC_body(h_ref, uh_ref, a0_ref, a1_ref, gh_ref, bh_ref, wu_ref, bu_ref,
                wa_ref, ba_ref, wb_ref, bb_ref, wv_ref, bv_ref,
                ho_ref, uho_ref, avo_ref, bho_ref):
    s = uh_ref[...] + a0_ref[...] + a1_ref[...]
    hn = jnp.maximum(_lnk(s, gh_ref[...], bh_ref[...]), 0.0)
    h = h_ref[...] + hn
    ho_ref[...] = h
    uho_ref[...] = _dot(h, wu_ref[...]) + bu_ref[...]
    avo_ref[:, 0:_H] = _dot(h, wa_ref[...]) + ba_ref[...]
    avo_ref[:, _H:2 * _H] = _dot(h, wv_ref[...]) + bv_ref[...]
    bho_ref[...] = _dot(h, wb_ref[...]) + bb_ref[...]


def _passC_last_body(h_ref, uh_ref, a0_ref, a1_ref, gh_ref, bh_ref, ho_ref):
    s = uh_ref[...] + a0_ref[...] + a1_ref[...]
    hn = jnp.maximum(_lnk(s, gh_ref[...], bh_ref[...]), 0.0)
    ho_ref[...] = h_ref[...] + hn


def _final_body(ee_ref, a_ref, s_ref, cw_ref, cb_ref, o_ref):
    y = jnp.maximum(ee_ref[...] * a_ref[...] + s_ref[...], 0.0)
    o_ref[...] = _dot(y, cw_ref[...]) + cb_ref[...]


def _full(shape):
    return pl.BlockSpec(shape, lambda i: (0,) * len(shape))


def _rows(blk, width):
    return pl.BlockSpec((blk, width), lambda i: (i, 0))


def _node_embed(x, dt64, ph64, wt, b):
    grid = (_N // _NBLK,)
    return pl.pallas_call(
        _node_embed_body,
        grid=grid,
        in_specs=[_rows(_NBLK, 2), _full((1, 64)), _full((1, 64)),
                  _full((_H, _H)), _full((1, _H))],
        out_specs=_rows(_NBLK, _H),
        out_shape=jax.ShapeDtypeStruct((_N, _H), _F32),
    )(x, dt64, ph64, wt, b)


def _edge_embed(e2, dt128, ph128, wt, b, w4, b4):
    grid = (_E // _EBLK,)
    return pl.pallas_call(
        _edge_embed_body,
        grid=grid,
        in_specs=[_rows(_EBLK, 1), _full((1, _H)), _full((1, _H)),
                  _full((_H, _H)), _full((1, _H)), _full((_H, _H)),
                  _full((1, _H))],
        out_specs=[_rows(_EBLK, _H), _rows(_EBLK, _H)],
        out_shape=[jax.ShapeDtypeStruct((_E, _H), _F32),
                   jax.ShapeDtypeStruct((_E, _H), _F32)],
    )(e2, dt128, ph128, wt, b, w4, b4)


def _nodeproj(h, wu, bu, wa, ba, wb, bb, wv, bv):
    grid = (_N // _NBLK,)
    wspecs = [_full((_H, _H)), _full((1, _H))] * 4
    return pl.pallas_call(
        _nodeproj_body,
        grid=grid,
        in_specs=[_rows(_NBLK, _H)] + wspecs,
        out_specs=[_rows(_NBLK, _H), _rows(_NBLK, 2 * _H), _rows(_NBLK, _H)],
        out_shape=[jax.ShapeDtypeStruct((_N, _H), _F32),
                   jax.ShapeDtypeStruct((_N, 2 * _H), _F32),
                   jax.ShapeDtypeStruct((_N, _H), _F32)],
    )(h, wu, bu, wa, ba, wb, bb, wv, bv)


def _passB(e_new, ee_in, ge, be, gp, bp, pw, pb2, w4, b4):
    grid = (_E // _EBLK,)
    return pl.pallas_call(
        _passB_body,
        grid=grid,
        in_specs=[_rows(_EBLK, _H), _rows(_EBLK, _H),
                  _full((1, _H)), _full((1, _H)), _full((1, _H)),
                  _full((1, _H)), _full((_H, _H)), _full((1, _H)),
                  _full((_H, _H)), _full((1, _H))],
        out_specs=[_rows(_EBLK, _H), _rows(_EBLK, _H)],
        out_shape=[jax.ShapeDtypeStruct((_E, _H), _F32),
                   jax.ShapeDtypeStruct((_E, _H), _F32)],
    )(e_new, ee_in, ge, be, gp, bp, pw, pb2, w4, b4)


def _passB_last(e_new, ee_in, ge, be, gp, bp, pw, pb2):
    grid = (_E // _EBLK,)
    return pl.pallas_call(
        _passB_last_body,
        grid=grid,
        in_specs=[_rows(_EBLK, _H), _rows(_EBLK, _H),
                  _full((1, _H)), _full((1, _H)), _full((1, _H)),
                  _full((1, _H)), _full((_H, _H)), _full((1, _H))],
        out_specs=[_rows(_EBLK, _H), _full((8, _H))],
        out_shape=[jax.ShapeDtypeStruct((_E, _H), _F32),
                   jax.ShapeDtypeStruct((8, _H), _F32)],
    )(e_new, ee_in, ge, be, gp, bp, pw, pb2)


def _passC(h, uh, agg, gh, bh, wu, bu, wa, ba, wb, bb, wv, bv):
    grid = (_N // _NBLK,)
    nblocks = _N // _NBLK
    a1_spec = pl.BlockSpec((_NBLK, _H), lambda i: (i + nblocks, 0))
    wspecs = [_full((_H, _H)), _full((1, _H))] * 4
    return pl.pallas_call(
        _passC_body,
        grid=grid,
        in_specs=[_rows(_NBLK, _H), _rows(_NBLK, _H), _rows(_NBLK, _H),
                  a1_spec, _full((1, _H)), _full((1, _H))] + wspecs,
        out_specs=[_rows(_NBLK, _H), _rows(_NBLK, _H), _rows(_NBLK, 2 * _H),
                   _rows(_NBLK, _H)],
        out_shape=[jax.ShapeDtypeStruct((_N, _H), _F32),
                   jax.ShapeDtypeStruct((_N, _H), _F32),
                   jax.ShapeDtypeStruct((_N, 2 * _H), _F32),
                   jax.ShapeDtypeStruct((_N, _H), _F32)],
    )(h, uh, agg, agg, gh, bh, wu, bu, wa, ba, wb, bb, wv, bv)


def _passC_last(h, uh, agg, gh, bh):
    grid = (_N // _NBLK,)
    nblocks = _N // _NBLK
    a1_spec = pl.BlockSpec((_NBLK, _H), lambda i: (i + nblocks, 0))
    return pl.pallas_call(
        _passC_last_body,
        grid=grid,
        in_specs=[_rows(_NBLK, _H), _rows(_NBLK, _H), _rows(_NBLK, _H),
                  a1_spec, _full((1, _H)), _full((1, _H))],
        out_specs=_rows(_NBLK, _H),
        out_shape=jax.ShapeDtypeStruct((_N, _H), _F32),
    )(h, uh, agg, agg, gh, bh)


def _final(ee, a, s, cw, cb):
    grid = (_E // _EBLK,)
    return pl.pallas_call(
        _final_body,
        grid=grid,
        in_specs=[_rows(_EBLK, _H), _full((1, _H)), _full((1, _H)),
                  _full((_H, 8)), _full((1, 8))],
        out_specs=_rows(_EBLK, 8),
        out_shape=jax.ShapeDtypeStruct((_E, 8), _F32),
    )(ee, a, s, cw, cb)


_ZROWS = 40
_NZCHUNKS = _N // _ZROWS


_NLOC = _NCHUNKS // _NW


@functools.lru_cache(maxsize=1)
def _build_sc_edge():
    mesh = plsc.VectorSubcoreMesh(core_axis_name="c", subcore_axis_name="s",
                                  num_cores=_NC, num_subcores=_NS)

    scratch = (
        [pltpu.VMEM((_CH,), jnp.int32) for _ in range(4)]
        + [pltpu.VMEM((_CH,), jnp.int32) for _ in range(4)]
        + [pltpu.VMEM((_CH, 2 * _H), _F32) for _ in range(2)]
        + [pltpu.VMEM((_CH, _H), _F32) for _ in range(2)]
        + [pltpu.VMEM((_CH, _H), _F32) for _ in range(2)]
        + [pltpu.VMEM_SHARED((_N, _H), _F32)]
        + [pltpu.SemaphoreType.DMA for _ in range(12)]
    )

    @functools.partial(
        pl.kernel,
        out_type=(
            jax.ShapeDtypeStruct((_E, _H), _F32),
            jax.ShapeDtypeStruct((_NC * _N, _H), _F32),
        ),
        mesh=mesh,
        scratch_types=scratch,
    )
    def sc_edge(av_hbm, bh_hbm, ce_hbm, dst_hbm, src_hbm,
                enew_hbm, agg_hbm,
                d0, d1, d2, d3, s0, s1, s2, s3,
                av0, av1, bh0, bh1, ce0, ce1, agg_sh,
                id0, id1, is0, is1, ga0, ga1, gb0, gb1, we0, we1, ws0, ws1):
        c = lax.axis_index("c")
        s = lax.axis_index("s")
        wid = s * _NC + c

        dstv = [d0, d1, d2, d3]
        srcv = [s0, s1, s2, s3]
        avv = [av0, av1]
        bhv = [bh0, bh1]
        cev = [ce0, ce1]
        isem_d = [id0, id1]
        isem_s = [is0, is1]
        gsem_a = [ga0, ga1]
        gsem_b = [gb0, gb1]
        wsem_e = [we0, we1]
        wsem_s = [ws0, ws1]

        zvec = jnp.zeros((16,), _F32)

        def zfill(r, carry):
            for i in range(_H // 16):
                ce0[r, pl.ds(i * 16, 16)] = zvec
            return carry

        lax.fori_loop(0, _ZROWS, zfill, 0)

        nz = (_NZCHUNKS - s + _NS - 1) // _NS

        def zcopy(k, carry):
            g = s + k * _NS
            pltpu.sync_copy(ce0, agg_sh.at[pl.ds(g * _ZROWS, _ZROWS)])
            return carry

        lax.fori_loop(0, nz, zcopy, 0)
        plsc.subcore_barrier()

        def base(i):
            return (wid + i * _NW) * _CH

        def issue_idx(i, slot, sync):
            b = base(i)
            if sync:
                pltpu.sync_copy(dst_hbm.at[pl.ds(b, _CH)], dstv[slot])
                pltpu.sync_copy(src_hbm.at[pl.ds(b, _CH)], srcv[slot])
            else:
                pltpu.async_copy(dst_hbm.at[pl.ds(b, _CH)], dstv[slot],
                                 isem_d[slot % 2])
                pltpu.async_copy(src_hbm.at[pl.ds(b, _CH)], srcv[slot],
                                 isem_s[slot % 2])

        def wait_idx(i, slot):
            b = base(i)
            pltpu.make_async_copy(dst_hbm.at[pl.ds(b, _CH)], dstv[slot],
                                  isem_d[slot % 2]).wait()
            pltpu.make_async_copy(src_hbm.at[pl.ds(b, _CH)], srcv[slot],
                                  isem_s[slot % 2]).wait()

        def issue_gathers(i, p, slot):
            b = base(i)
            pltpu.async_copy(av_hbm.at[dstv[slot]], avv[p], gsem_a[p])
            pltpu.async_copy(bh_hbm.at[srcv[slot]], bhv[p], gsem_b[p])
            pltpu.async_copy(ce_hbm.at[pl.ds(b, _CH)], cev[p], gsem_a[p])

        def wait_gathers(i, p, slot):
            b = base(i)
            pltpu.make_async_copy(av_hbm.at[dstv[slot]], avv[p],
                                  gsem_a[p]).wait()
            pltpu.make_async_copy(bh_hbm.at[srcv[slot]], bhv[p],
                                  gsem_b[p]).wait()
            pltpu.make_async_copy(ce_hbm.at[pl.ds(b, _CH)], cev[p],
                                  gsem_a[p]).wait()

        def issue_writes(i, p, slot):
            b = base(i)
            pltpu.async_copy(cev[p], enew_hbm.at[pl.ds(b, _CH)], wsem_e[p])
            pltpu.async_copy(bhv[p], agg_sh.at[srcv[slot]], wsem_s[p],
                             add=True)

        def wait_writes(i, p, slot):
            b = base(i)
            pltpu.make_async_copy(cev[p], enew_hbm.at[pl.ds(b, _CH)],
                                  wsem_e[p]).wait()
            pltpu.make_async_copy(bhv[p], agg_sh.at[srcv[slot]],
                                  wsem_s[p]).wait()

        def compute(p):
            def row(j, carry2):
                ens = []
                gvs = []
                for i2 in range(_H // 16):
                    a = avv[p][j, pl.ds(i2 * 16, 16)]
                    v = avv[p][j, pl.ds(_H + i2 * 16, 16)]
                    bb = bhv[p][j, pl.ds(i2 * 16, 16)]
                    cc = cev[p][j, pl.ds(i2 * 16, 16)]
                    en = a + bb + cc
                    ens.append(en)
                    gvs.append(v / (1.0 + jnp.exp(-en)))
                for i2 in range(_H // 16):
                    cev[p][j, pl.ds(i2 * 16, 16)] = ens[i2]
                    bhv[p][j, pl.ds(i2 * 16, 16)] = gvs[i2]
                return carry2

            lax.fori_loop(0, _CH, row, 0, unroll=2)

        issue_idx(0, 0, sync=True)
        issue_idx(1, 1, sync=False)
        issue_gathers(0, 0, 0)

        def quad(k, carry):
            for j in range(4):
                i = 4 * k + j
                p = j % 2

                @pl.when(jnp.logical_and(i >= 1, i <= _NLOC))
                def _():
                    wait_writes(i - 1, 1 - p, (j - 1) % 4)

                @pl.when(i + 2 <= _NLOC - 1)
                def _():
                    issue_idx(i + 2, (j + 2) % 4, sync=False)

                @pl.when(i + 1 <= _NLOC - 1)
                def _():
                    wait_idx(i + 1, (j + 1) % 4)
                    issue_gathers(i + 1, 1 - p, (j + 1) % 4)

                @pl.when(i <= _NLOC - 1)
                def _():
                    wait_gathers(i, p, j)
                    compute(p)
                    issue_writes(i, p, j)
            return carry

        lax.fori_loop(0, (_NLOC + 4) // 4, quad, 0)
        plsc.subcore_barrier()

        def ocopy(k, carry):
            g = s + k * _NS
            pltpu.sync_copy(agg_sh.at[pl.ds(g * _ZROWS, _ZROWS)],
                            agg_hbm.at[pl.ds(c * _N + g * _ZROWS, _ZROWS)])
            return carry

        lax.fori_loop(0, nz, ocopy, 0)

    return sc_edge


def _sc_edge(av, bh, ce, dst, src):
    return _build_sc_edge()(av, bh, ce, dst, src)


def kernel(task, x, e, edge_index, node_W, node_b, edge_W, edge_b, lin_W,
           lin_b, ln_h_g, ln_h_b, ln_e_g, ln_e_b, plo_g, plo_b, plo_W,
           plo_b2, gn_g, gn_b, conv_W, conv_b):
    src = edge_index[0]
    dst = edge_index[1]

    i64 = np.arange(64, dtype=np.float64)
    dt64 = (10000.0 ** (2.0 * np.floor(i64 / 2.0) / 64.0))
    dt64 = jnp.asarray(dt64.reshape(1, 64), dtype=_F32)
    ph64 = jnp.asarray(((i64 % 2.0) * (np.pi / 2.0)).reshape(1, 64), _F32)
    i128 = np.arange(128, dtype=np.float64)
    dt128 = (10000.0 ** (2.0 * np.floor(i128 / 2.0) / 128.0))
    dt128 = jnp.asarray(dt128.reshape(1, 128), dtype=_F32)
    ph128 = jnp.asarray(((i128 % 2.0) * (np.pi / 2.0)).reshape(1, 128), _F32)

    r1 = lambda a: a.reshape(1, _H)
    w4t = [lin_W[l, 4].T for l in range(4)]
    b4 = [r1(lin_b[l, 4]) for l in range(4)]
    wut = [lin_W[l, 0].T for l in range(4)]
    but = [r1(lin_b[l, 0]) for l in range(4)]
    wvt = [lin_W[l, 1].T for l in range(4)]
    bvt = [r1(lin_b[l, 1]) for l in range(4)]
    wat = [lin_W[l, 2].T for l in range(4)]
    bat = [r1(lin_b[l, 2]) for l in range(4)]
    wbt = [lin_W[l, 3].T for l in range(4)]
    bbt = [r1(lin_b[l, 3]) for l in range(4)]
    pwt = [plo_W[l].T for l in range(4)]

    h = _node_embed(x, dt64, ph64, node_W.T, r1(node_b))
    ee, ce = _edge_embed(e.reshape(_E, 1), dt128, ph128, edge_W.T,
                         r1(edge_b), w4t[0], b4[0])
    uh, av, bh = _nodeproj(h, wut[0], but[0], wat[0], bat[0],
                           wbt[0], bbt[0], wvt[0], bvt[0])

    moments = None
    for l in range(4):
        e_new, agg = _sc_edge(av, bh, ce, dst, src)
        if l < 3:
            ee, ce = _passB(e_new, ee, r1(ln_e_g[l]), r1(ln_e_b[l]),
                            r1(plo_g[l]), r1(plo_b[l]), pwt[l],
                            r1(plo_b2[l]), w4t[l + 1], b4[l + 1])
            h, uh, av, bh = _passC(h, uh, agg, r1(ln_h_g[l]), r1(ln_h_b[l]),
                                   wut[l + 1], but[l + 1], wat[l + 1],
                                   bat[l + 1], wbt[l + 1], bbt[l + 1],
                                   wvt[l + 1], bvt[l + 1])
        else:
            ee, moments = _passB_last(e_new, ee, r1(ln_e_g[l]), r1(ln_e_b[l]),
                                      r1(plo_g[l]), r1(plo_b[l]), pwt[l],
                                      r1(plo_b2[l]))
            h = _passC_last(h, uh, agg, r1(ln_h_g[l]), r1(ln_h_b[l]))

    csum = moments[0]
    csumsq = moments[1]
    cnt = np.float32(4 * _E)
    gmean = csum.reshape(32, 4).sum(axis=1) / cnt
    gsq = csumsq.reshape(32, 4).sum(axis=1) / cnt
    gvar = gsq - gmean * gmean
    ginv = 1.0 / jnp.sqrt(gvar + 1e-5)
    mean128 = jnp.repeat(gmean, 4)
    inv128 = jnp.repeat(ginv, 4)
    a = (inv128 * gn_g).reshape(1, _H)
    sh = (gn_b - mean128 * inv128 * gn_g).reshape(1, _H)
    cw = jnp.zeros((_H, 8), _F32).at[:, 0:2].set(conv_W.T)
    cb = jnp.zeros((1, 8), _F32).at[0, 0:2].set(conv_b)
    e_out8 = _final(ee, a, sh, cw, cb)
    e_out = e_out8[:, 0:2]
    return h, e_out
